# baseline (speedup 1.0000x reference)
import functools

import jax
import jax.numpy as jnp
from jax import lax
from jax.experimental import pallas as pl
from jax.experimental.pallas import tpu as pltpu
from jax.experimental.pallas import tpu_sc as plsc

B, C, H, W = 4, 256, 200, 176
HW = H * W
N = 100000
BLK = 192
LAST_BASE = N - BLK
K0, K1 = 16, 17
NITER = (max(K0, K1) + 1) // 2


def _make_gather():
    mesh = plsc.VectorSubcoreMesh(core_axis_name="c", subcore_axis_name="s")

    @functools.partial(
        pl.kernel,
        mesh=mesh,
        out_type=jax.ShapeDtypeStruct((N, C), jnp.float32),
        scratch_types=[
            pltpu.VMEM((BLK,), jnp.int32), pltpu.VMEM((BLK,), jnp.int32),
            pltpu.VMEM((BLK, C), jnp.float32),
            pltpu.VMEM((BLK, C), jnp.float32),
            pltpu.SemaphoreType.DMA, pltpu.SemaphoreType.DMA,
            pltpu.SemaphoreType.DMA, pltpu.SemaphoreType.DMA,
            pltpu.SemaphoreType.DMA, pltpu.SemaphoreType.DMA,
        ],
    )
    def gather_kernel(table_hbm, ridx_hbm, out_hbm,
                      idx0, idx1, rows0, rows1,
                      sem_i0, sem_i1, sem_g0, sem_g1, sem_w0, sem_w1):
        cid = lax.axis_index("c")
        sid = lax.axis_index("s")
        nblk = jnp.where(cid == 0, K0, K1)
        g0 = jnp.where(cid == 0, sid, 16 * K0 + sid)
        idx, rows = (idx0, idx1), (rows0, rows1)
        sem_i, sem_g, sem_w = (sem_i0, sem_i1), (sem_g0, sem_g1), (sem_w0, sem_w1)

        def blk_of(jj, s):
            jl = 2 * jj + s
            g = g0 + 16 * jl
            base = pl.multiple_of(jnp.minimum(g * BLK, LAST_BASE), 8)
            return jl, base

        def fire_idx(s, base):
            pltpu.async_copy(ridx_hbm.at[pl.ds(base, BLK)], idx[s], sem_i[s])

        def wait_idx(s, base):
            pltpu.make_async_copy(
                ridx_hbm.at[pl.ds(base, BLK)], idx[s], sem_i[s]).wait()

        for s in range(2):
            jl, base = blk_of(0, s)

            @pl.when(jl < nblk)
            def _(s=s, base=base):
                fire_idx(s, base)

        def body(jj, carry):
            for s in range(2):
                jl, base = blk_of(jj, s)

                @pl.when(jl < nblk)
                def _(s=s, base=base):
                    wait_idx(s, base)

                    @pl.when(jj > 0)
                    def _():
                        pltpu.make_async_copy(
                            rows[s], out_hbm.at[pl.ds(0, BLK)], sem_w[s]
                        ).wait()
                    pltpu.async_copy(table_hbm.at[idx[s]], rows[s], sem_g[s])

            for s in range(2):
                jl, base = blk_of(jj, s)

                @pl.when(jl < nblk)
                def _(s=s, base=base):
                    pltpu.make_async_copy(
                        table_hbm.at[idx[s]], rows[s], sem_g[s]).wait()
                    pltpu.async_copy(
                        rows[s], out_hbm.at[pl.ds(base, BLK)], sem_w[s])
                    jl2, base2 = blk_of(jj + 1, s)

                    @pl.when(jl2 < nblk)
                    def _():
                        fire_idx(s, base2)

            return carry

        lax.fori_loop(0, NITER, body, 0)
        for s in range(2):
            pltpu.make_async_copy(
                rows[s], out_hbm.at[pl.ds(0, BLK)], sem_w[s]).wait()

    return gather_kernel


_gather = _make_gather()


def kernel(spatial_features_2d, voxel_coords):
    table = jnp.transpose(spatial_features_2d, (0, 2, 3, 1)).reshape(B * HW, C)
    vc = voxel_coords.astype(jnp.int32)
    ridx = vc[:, 0] * HW + vc[:, 2] * W + vc[:, 3]
    return _gather(table, ridx)


# speedup vs baseline: 1.0392x; 1.0392x over previous
import functools

import jax
import jax.numpy as jnp
from jax import lax
from jax.experimental import pallas as pl
from jax.experimental.pallas import tpu as pltpu
from jax.experimental.pallas import tpu_sc as plsc

B, C, H, W = 4, 256, 200, 176
HW = H * W
N = 100000
BLK = 128
LAST_BASE = N - BLK
K0, K1 = 24, 25
NSLOT = 3
NITER = (max(K0, K1) + NSLOT - 1) // NSLOT


def _make_gather():
    mesh = plsc.VectorSubcoreMesh(core_axis_name="c", subcore_axis_name="s")

    @functools.partial(
        pl.kernel,
        mesh=mesh,
        out_type=jax.ShapeDtypeStruct((N, C), jnp.float32),
        scratch_types=[
            *([pltpu.VMEM((BLK,), jnp.int32)] * 12),
            *([pltpu.VMEM((BLK, C), jnp.float32)] * 3),
            *([pltpu.SemaphoreType.DMA] * 9),
        ],
    )
    def gather_kernel(table_hbm, coords_flat_hbm, out_hbm,
                      bv0, yv0, xv0, bv1, yv1, xv1, bv2, yv2, xv2,
                      idx0, idx1, idx2, rows0, rows1, rows2,
                      sem_c0, sem_c1, sem_c2, sem_g0, sem_g1, sem_g2,
                      sem_w0, sem_w1, sem_w2):
        cid = lax.axis_index("c")
        sid = lax.axis_index("s")
        nblk = jnp.where(cid == 0, K0, K1)
        g0 = jnp.where(cid == 0, sid, 16 * K0 + sid)
        bv, yv, xv = (bv0, bv1, bv2), (yv0, yv1, yv2), (xv0, xv1, xv2)
        idx, rows = (idx0, idx1, idx2), (rows0, rows1, rows2)
        sem_c = (sem_c0, sem_c1, sem_c2)
        sem_g = (sem_g0, sem_g1, sem_g2)
        sem_w = (sem_w0, sem_w1, sem_w2)

        def blk_of(jj, s):
            jl = NSLOT * jj + s
            g = g0 + 16 * jl
            base = pl.multiple_of(jnp.minimum(g * BLK, LAST_BASE), 8)
            return jl, base

        def fire_coords(s, base):
            pltpu.async_copy(coords_flat_hbm.at[pl.ds(base, BLK)], bv[s], sem_c[s])
            pltpu.async_copy(coords_flat_hbm.at[pl.ds(2 * N + base, BLK)], yv[s], sem_c[s])
            pltpu.async_copy(coords_flat_hbm.at[pl.ds(3 * N + base, BLK)], xv[s], sem_c[s])

        def wait_coords(s, base):
            pltpu.make_async_copy(
                coords_flat_hbm.at[pl.ds(base, BLK)], bv[s], sem_c[s]).wait()
            pltpu.make_async_copy(
                coords_flat_hbm.at[pl.ds(2 * N + base, BLK)], yv[s], sem_c[s]).wait()
            pltpu.make_async_copy(
                coords_flat_hbm.at[pl.ds(3 * N + base, BLK)], xv[s], sem_c[s]).wait()

        for s in range(NSLOT):
            jl, base = blk_of(0, s)

            @pl.when(jl < nblk)
            def _(s=s, base=base):
                fire_coords(s, base)

        def body(jj, carry):
            for s in range(NSLOT):
                jl, base = blk_of(jj, s)

                @pl.when(jl < nblk)
                def _(s=s, base=base):
                    wait_coords(s, base)
                    for i in range(BLK // 16):
                        sl = pl.ds(i * 16, 16)
                        idx[s][sl] = bv[s][sl] * HW + yv[s][sl] * W + xv[s][sl]

                    @pl.when(jj > 0)
                    def _():
                        pltpu.make_async_copy(
                            rows[s], out_hbm.at[pl.ds(0, BLK)], sem_w[s]
                        ).wait()
                    pltpu.async_copy(table_hbm.at[idx[s]], rows[s], sem_g[s])

            for s in range(NSLOT):
                jl, base = blk_of(jj, s)

                @pl.when(jl < nblk)
                def _(s=s, base=base):
                    pltpu.make_async_copy(
                        table_hbm.at[idx[s]], rows[s], sem_g[s]).wait()
                    pltpu.async_copy(
                        rows[s], out_hbm.at[pl.ds(base, BLK)], sem_w[s])
                    jl2, base2 = blk_of(jj + 1, s)

                    @pl.when(jl2 < nblk)
                    def _():
                        fire_coords(s, base2)

            return carry

        lax.fori_loop(0, NITER, body, 0)
        for s in range(NSLOT):
            pltpu.make_async_copy(
                rows[s], out_hbm.at[pl.ds(0, BLK)], sem_w[s]).wait()

    return gather_kernel


_gather = _make_gather()


def kernel(spatial_features_2d, voxel_coords):
    table = jnp.transpose(spatial_features_2d, (0, 2, 3, 1)).reshape(B * HW, C)
    vc_flat = jnp.transpose(voxel_coords.astype(jnp.int32), (1, 0)).reshape(4 * N)
    return _gather(table, vc_flat)
